# baseline (speedup 1.0000x reference)
import functools

import jax
import jax.numpy as jnp
from jax import lax
from jax.experimental import pallas as pl
from jax.experimental.pallas import tpu as pltpu
from jax.experimental.pallas import tpu_sc as plsc

B = 128
N = 32768
K = 64
L = 16
NV = N // L
NC = 2
NS = 16
NW = NC * NS
ROWS_PER_W = B // NW
CMAX = 1024
CV = CMAX // L
NEG = float("-inf")
BIG = 1 << 30
PR = 8


def _splat_f(x):
    return jnp.full((L,), x, dtype=jnp.float32)


def _splat_i(x):
    return jnp.full((L,), x, dtype=jnp.int32)


def _prep_body(x_ref, o_ref):
    x = x_ref[...]
    cms = []
    for g in range(L):
        m = x[:, g * (L * 128):g * (L * 128) + 128]
        for j in range(1, L):
            lo = g * (L * 128) + j * 128
            m = jnp.maximum(m, x[:, lo:lo + 128])
        cms.append(m)
        o_ref[:, g * 128:(g + 1) * 128] = m
    colmax = cms[0]
    for g in range(1, L):
        colmax = jnp.maximum(colmax, cms[g])
    pairmax = jnp.maximum(colmax, pltpu.roll(colmax, 64, axis=1))
    thr = jnp.min(pairmax, axis=1)
    o_ref[:, NV:] = jnp.broadcast_to(thr[:, None], (PR, L))


@functools.lru_cache(maxsize=1)
def _prep_call():
    return pl.pallas_call(
        _prep_body,
        grid=(B // PR,),
        in_specs=[pl.BlockSpec((PR, N), lambda i: (i, 0))],
        out_specs=pl.BlockSpec((PR, NV + L), lambda i: (i, 0)),
        out_shape=jax.ShapeDtypeStruct((B, NV + L), jnp.float32),
    )


def _select_slow(vload, iload, vkill, nvec, ntab, tval, tidx,
                 ovbuf, oibuf, lane0, iota):
    neg16 = _splat_f(NEG)
    big16 = _splat_i(BIG)

    def vec_best(v):
        x = vload(v)
        m = jnp.max(x)
        ti = jnp.min(jnp.where(x == _splat_f(m), iload(v), big16))
        return m, ti

    def t_build(v, _):
        m, ti = vec_best(v)
        plsc.store_scatter(tval, [_splat_i(v)], _splat_f(m), mask=lane0)
        plsc.store_scatter(tidx, [_splat_i(v)], _splat_i(ti), mask=lane0)
        return 0

    lax.fori_loop(0, nvec, t_build, 0)

    def k_body(k, _):
        def scan(t, carry):
            lmax, lidx, lpos = carry
            x = tval[pl.ds(t * L, L)]
            xi = tidx[pl.ds(t * L, L)]
            gt = (x > lmax) | ((x == lmax) & (xi < lidx))
            lmax = jnp.where(gt, x, lmax)
            lidx = jnp.where(gt, xi, lidx)
            lpos = jnp.where(gt, _splat_i(t * L) + iota, lpos)
            return (lmax, lidx, lpos)

        lmax, lidx, lpos = lax.fori_loop(
            0, ntab, scan, (_splat_f(NEG), big16, _splat_i(0)))
        m = jnp.max(lmax)
        msp = _splat_f(m)
        is_m = lmax == msp
        gidx = jnp.min(jnp.where(is_m, lidx, big16))
        gisp = _splat_i(gidx)
        p = jnp.min(jnp.where(is_m & (lidx == gisp), lpos, big16))
        plsc.store_scatter(ovbuf, [_splat_i(k)], msp, mask=lane0)
        plsc.store_scatter(oibuf, [_splat_i(k)], gisp, mask=lane0)
        q = jnp.min(jnp.where(iload(p) == gisp, iota, big16))
        vkill(_splat_i(p * L + q))
        m2, ti2 = vec_best(p)
        plsc.store_scatter(tval, [_splat_i(p)], _splat_f(m2), mask=lane0)
        plsc.store_scatter(tidx, [_splat_i(p)], _splat_i(ti2), mask=lane0)
        return 0

    lax.fori_loop(0, K, k_body, 0)


def _extract_rows(slots, lane0, iota):
    neg16 = _splat_f(NEG)
    big16 = _splat_i(BIG)

    def k_body(k, _):
        for (cval, cidx, tval, tidx, ovbuf, oibuf) in slots:
            lmax = tval[pl.ds(0, L)]
            lidx = tidx[pl.ds(0, L)]
            lpos = iota
            for t in range(1, CV // L):
                x = tval[pl.ds(t * L, L)]
                xi = tidx[pl.ds(t * L, L)]
                gt = (x > lmax) | ((x == lmax) & (xi < lidx))
                lmax = jnp.where(gt, x, lmax)
                lidx = jnp.where(gt, xi, lidx)
                lpos = jnp.where(gt, _splat_i(t * L) + iota, lpos)
            m = jnp.max(lmax)
            msp = _splat_f(m)
            is_m = lmax == msp
            gidx = jnp.min(jnp.where(is_m, lidx, big16))
            gisp = _splat_i(gidx)
            p = jnp.min(jnp.where(is_m & (lidx == gisp), lpos, big16))
            c = cval[pl.ds(p * L, L)]
            ci = cidx[pl.ds(p * L, L)]
            c2 = jnp.where(ci == gisp, neg16, c)
            cval[pl.ds(p * L, L)] = c2
            m2 = jnp.max(c2)
            ti2 = jnp.min(jnp.where(c2 == _splat_f(m2), ci, big16))
            plsc.store_scatter(tval, [_splat_i(p)], _splat_f(m2), mask=lane0)
            plsc.store_scatter(tidx, [_splat_i(p)], _splat_i(ti2), mask=lane0)
            plsc.store_scatter(ovbuf, [_splat_i(k)], msp, mask=lane0)
            plsc.store_scatter(oibuf, [_splat_i(k)], gisp, mask=lane0)
        return 0

    lax.fori_loop(0, K, k_body, 0)


def _topk_body(acc_hbm, vmt_hbm, outv_hbm, outi_hbm,
               rowbuf0, rowbuf1, vmbuf0, vmbuf1, vidbuf,
               bcnt, boffs, ccnt, coffs, basbuf,
               cval0, cval1, cval2, cval3, cidx0, cidx1, cidx2, cidx3,
               tval0, tval1, tval2, tval3, tidx0, tidx1, tidx2, tidx3,
               tfv, tfi,
               ovbuf0, ovbuf1, ovbuf2, ovbuf3, oibuf0, oibuf1, oibuf2, oibuf3,
               sem0, sem1, semv0, semv1):
    wid = lax.axis_index("s") * NC + lax.axis_index("c")
    base_row = wid * ROWS_PER_W
    iota = lax.broadcasted_iota(jnp.int32, (L,), 0)
    lane0 = iota == 0
    neg16 = _splat_f(NEG)
    big16 = _splat_i(BIG)
    zero16 = _splat_i(0)
    lim = _splat_i(CMAX)
    iota128 = iota * 128

    rbufs = (rowbuf0, rowbuf1)
    vbufs = (vmbuf0, vmbuf1)
    rsems = (sem0, sem1)
    vsems = (semv0, semv1)
    cvals = (cval0, cval1, cval2, cval3)
    cidxs = (cidx0, cidx1, cidx2, cidx3)
    tvals = (tval0, tval1, tval2, tval3)
    tidxs = (tidx0, tidx1, tidx2, tidx3)
    ovbufs = (ovbuf0, ovbuf1, ovbuf2, ovbuf3)
    oibufs = (oibuf0, oibuf1, oibuf2, oibuf3)

    rh = [None, None]
    vh = [None, None]
    rh[0] = pltpu.async_copy(acc_hbm.at[base_row], rbufs[0], rsems[0])
    vh[0] = pltpu.async_copy(vmt_hbm.at[base_row], vbufs[0], vsems[0])
    fasts = []
    for r in range(ROWS_PER_W):
        cur = r % 2
        nxt = (r + 1) % 2
        if r + 1 < ROWS_PER_W:
            rh[nxt] = pltpu.async_copy(
                acc_hbm.at[base_row + (r + 1)], rbufs[nxt], rsems[nxt])
            vh[nxt] = pltpu.async_copy(
                vmt_hbm.at[base_row + (r + 1)], vbufs[nxt], vsems[nxt])
        rh[cur].wait()
        vh[cur].wait()
        row = rbufs[cur]
        vmb = vbufs[cur]
        thr_s = vmb[pl.ds(NV, L)]
        cval = cvals[r]
        cidx = cidxs[r]
        tval = tvals[r]
        tidx = tidxs[r]

        def b_count(i, _):
            msk = vmb[pl.ds(i * L, L)] >= thr_s
            plsc.store_scatter(bcnt, [_splat_i(i)],
                               plsc.all_reduce_population_count(msk),
                               mask=lane0)
            return 0

        lax.fori_loop(0, NV // L // L, lambda t, _: [
            b_count(t * L + u, 0) for u in range(L)][-1], 0)

        carry = zero16
        for t in range(NV // L // L):
            c = bcnt[pl.ds(t * L, L)]
            cs = plsc.cumsum(c)
            boffs[pl.ds(t * L, L)] = carry + cs - c
            carry = carry + _splat_i(jnp.max(cs))
        nq = jnp.max(carry)

        def b_scat(i, _):
            msk = vmb[pl.ds(i * L, L)] >= thr_s
            pfx = plsc.cumsum(msk.astype(jnp.int32))
            off = plsc.load_gather(boffs, [_splat_i(i)])
            tgt = off + pfx - 1
            ok = msk & (tgt < lim)
            plsc.store_scatter(vidbuf, [tgt], iota + i * L, mask=ok)
            return 0

        lax.fori_loop(0, NV // L // L, lambda t, _: [
            b_scat(t * L + u, 0) for u in range(L)][-1], 0)

        nqc = jnp.minimum(nq, CMAX)
        nqv = (nqc + (L - 1)) // L
        nq_s = _splat_i(nqc)

        def c_bases(t, _):
            qc = jnp.bitwise_and(vidbuf[pl.ds(t * L, L)], NV - 1)
            bas = lax.shift_left(lax.shift_right_logical(qc, 7), 11) \
                + jnp.bitwise_and(qc, 127)
            basbuf[pl.ds(t * L, L)] = bas
            return 0

        lax.fori_loop(0, nqv, c_bases, 0)

        def c_count(t, _):
            for u in range(L):
                pln = t * L + u
                idxv = plsc.load_gather(basbuf, [_splat_i(pln)]) + iota128
                x = plsc.load_gather(row, [idxv])
                msk = (x >= thr_s) & (_splat_i(pln) < nq_s)
                plsc.store_scatter(ccnt, [_splat_i(pln)],
                                   plsc.all_reduce_population_count(msk),
                                   mask=lane0)
            return 0

        lax.fori_loop(0, nqv, c_count, 0)

        def c_prefix(t, carry):
            c = ccnt[pl.ds(t * L, L)]
            cs = plsc.cumsum(c)
            coffs[pl.ds(t * L, L)] = carry + cs - c
            return carry + _splat_i(jnp.max(cs))

        cntv = lax.fori_loop(0, nqv, c_prefix, zero16)
        cnt = jnp.max(cntv)

        for t in range(CV + 1):
            cval[pl.ds(t * L, L)] = neg16
            cidx[pl.ds(t * L, L)] = big16

        def c_scat(t, _):
            for u in range(L):
                pln = t * L + u
                idxv = plsc.load_gather(basbuf, [_splat_i(pln)]) + iota128
                x = plsc.load_gather(row, [idxv])
                msk = (x >= thr_s) & (_splat_i(pln) < nq_s)
                pfx = plsc.cumsum(msk.astype(jnp.int32))
                off = plsc.load_gather(coffs, [_splat_i(pln)])
                tgt = off + pfx - 1
                ok = msk & (tgt < lim)
                plsc.store_scatter(cval, [tgt], x, mask=ok)
                plsc.store_scatter(cidx, [tgt], idxv, mask=ok)
            return 0

        lax.fori_loop(0, nqv, c_scat, 0)

        for t in range(CV // L):
            tval[pl.ds(t * L, L)] = neg16
            tidx[pl.ds(t * L, L)] = big16

        def d_build(v, _):
            c = cval[pl.ds(v * L, L)]
            ci = cidx[pl.ds(v * L, L)]
            m = jnp.max(c)
            ti = jnp.min(jnp.where(c == _splat_f(m), ci, big16))
            plsc.store_scatter(tval, [_splat_i(v)], _splat_f(m), mask=lane0)
            plsc.store_scatter(tidx, [_splat_i(v)], _splat_i(ti), mask=lane0)
            return 0

        nvec = (jnp.minimum(cnt, CMAX) + (L - 1)) // L
        lax.fori_loop(0, nvec, d_build, 0)

        fast = (nq <= CMAX) & (cnt <= CMAX)
        fasts.append(fast)

        def fallback(_):
            _select_slow(
                vload=lambda p: row[pl.ds(p * L, L)],
                iload=lambda p: _splat_i(p * L) + iota,
                vkill=lambda gsp: plsc.store_scatter(
                    row, [gsp], neg16, mask=lane0),
                nvec=NV, ntab=NV // L, tval=tfv, tidx=tfi,
                ovbuf=ovbufs[r], oibuf=oibufs[r], lane0=lane0, iota=iota)
            return 0

        lax.cond(fast, lambda _: 0, fallback, 0)

    slots = [(cvals[r], cidxs[r], tvals[r], tidxs[r], ovbufs[r], oibufs[r])
             for r in range(ROWS_PER_W)]
    all_fast = fasts[0] & fasts[1] & fasts[2] & fasts[3]

    def e_all(_):
        _extract_rows(slots, lane0, iota)
        return 0

    def e_some(_):
        for r in range(ROWS_PER_W):
            lax.cond(fasts[r],
                     lambda _, rr=r: (_extract_rows([slots[rr]], lane0, iota),
                                      0)[1],
                     lambda _: 0, 0)
        return 0

    lax.cond(all_fast, e_all, e_some, 0)

    for r in range(ROWS_PER_W):
        pltpu.sync_copy(ovbufs[r], outv_hbm.at[base_row + r])
        pltpu.sync_copy(oibufs[r], outi_hbm.at[base_row + r])


@functools.lru_cache(maxsize=1)
def _topk_call():
    cbuf_f = pltpu.VMEM((CMAX + L,), jnp.float32)
    cbuf_i = pltpu.VMEM((CMAX + L,), jnp.int32)
    tbuf_f = pltpu.VMEM((CV,), jnp.float32)
    tbuf_i = pltpu.VMEM((CV,), jnp.int32)
    obuf_f = pltpu.VMEM((K,), jnp.float32)
    obuf_i = pltpu.VMEM((K,), jnp.int32)
    return functools.partial(
        pl.kernel,
        out_type=[
            jax.ShapeDtypeStruct((B, K), jnp.float32),
            jax.ShapeDtypeStruct((B, K), jnp.int32),
        ],
        mesh=plsc.VectorSubcoreMesh(core_axis_name="c", subcore_axis_name="s"),
        compiler_params=pltpu.CompilerParams(needs_layout_passes=False),
        scratch_types=[
            pltpu.VMEM((N,), jnp.float32),
            pltpu.VMEM((N,), jnp.float32),
            pltpu.VMEM((NV + L,), jnp.float32),
            pltpu.VMEM((NV + L,), jnp.float32),
            pltpu.VMEM((CMAX + L,), jnp.int32),
            pltpu.VMEM((NV // L,), jnp.int32),
            pltpu.VMEM((NV // L,), jnp.int32),
            pltpu.VMEM((CMAX,), jnp.int32),
            pltpu.VMEM((CMAX,), jnp.int32),
            pltpu.VMEM((CMAX + L,), jnp.int32),
            cbuf_f, cbuf_f, cbuf_f, cbuf_f,
            cbuf_i, cbuf_i, cbuf_i, cbuf_i,
            tbuf_f, tbuf_f, tbuf_f, tbuf_f,
            tbuf_i, tbuf_i, tbuf_i, tbuf_i,
            pltpu.VMEM((NV,), jnp.float32),
            pltpu.VMEM((NV,), jnp.int32),
            obuf_f, obuf_f, obuf_f, obuf_f,
            obuf_i, obuf_i, obuf_i, obuf_i,
            pltpu.SemaphoreType.DMA,
            pltpu.SemaphoreType.DMA,
            pltpu.SemaphoreType.DMA,
            pltpu.SemaphoreType.DMA,
        ],
    )(_topk_body)


def kernel(accumulated):
    vmt = _prep_call()(accumulated)
    topk_vals, topk_idx = _topk_call()(accumulated, vmt)
    return (topk_vals, topk_idx, accumulated)


# speedup vs baseline: 1.4267x; 1.4267x over previous
import functools

import jax
import jax.numpy as jnp
from jax import lax
from jax.experimental import pallas as pl
from jax.experimental.pallas import tpu as pltpu
from jax.experimental.pallas import tpu_sc as plsc

B = 128
N = 32768
K = 64
L = 16
NV = N // L
NC = 2
NS = 16
NW = NC * NS
ROWS_PER_W = B // NW
CMAX = 4096
NEG = float("-inf")
BIG = 1 << 30


def _splat_f(x):
    return jnp.full((L,), x, dtype=jnp.float32)


def _splat_i(x):
    return jnp.full((L,), x, dtype=jnp.int32)


SU = 4


def _select_topk(val_load, val_kill, idx_of, ngroups, ovbuf, oibuf, lane0, iota):
    def k_body(k, _):
        def scan(jg, carry):
            lmax, lpos = carry
            for u in range(SU):
                j = jg * SU + u
                x = val_load(j)
                gt = x > lmax
                lmax = jnp.maximum(lmax, x)
                lpos = jnp.where(gt, _splat_i(j * L) + iota, lpos)
            return (lmax, lpos)

        lmax, lpos = lax.fori_loop(
            0, ngroups, scan, (_splat_f(NEG), _splat_i(0)))
        m = jnp.max(lmax)
        msp = _splat_f(m)
        cand = jnp.where(lmax == msp, lpos, BIG)
        found = jnp.min(cand)
        fsp = _splat_i(found)
        plsc.store_scatter(ovbuf, [_splat_i(k)], msp, mask=lane0)
        plsc.store_scatter(oibuf, [_splat_i(k)], idx_of(fsp), mask=lane0)
        val_kill(fsp)
        return 0

    lax.fori_loop(0, K, k_body, 0)


def _topk_body(acc_hbm, outv_hbm, outi_hbm,
               rowbuf0, rowbuf1, cval, cidx, ovbuf, oibuf, sem0, sem1):
    wid = lax.axis_index("s") * NC + lax.axis_index("c")
    base_row = wid * ROWS_PER_W
    sems = (sem0, sem1)
    iota = lax.broadcasted_iota(jnp.int32, (L,), 0)
    lane0 = iota == 0
    neg16 = _splat_f(NEG)

    bufs = (rowbuf0, rowbuf1)
    handles = [None, None]
    handles[0] = pltpu.async_copy(acc_hbm.at[base_row], bufs[0], sems[0])
    for r in range(ROWS_PER_W):
        cur = r % 2
        nxt = (r + 1) % 2
        if r + 1 < ROWS_PER_W:
            handles[nxt] = pltpu.async_copy(
                acc_hbm.at[base_row + (r + 1)], bufs[nxt], sems[nxt])
        handles[cur].wait()
        row = bufs[cur]

        QB = NV // 4
        U1 = 4

        def p1(i, accs):
            a0, a1, a2, a3 = accs
            for u in range(U1):
                a0 = jnp.maximum(a0, row[pl.ds((i * U1 + u) * L, L)])
                a1 = jnp.maximum(a1, row[pl.ds((QB + i * U1 + u) * L, L)])
                a2 = jnp.maximum(a2, row[pl.ds((2 * QB + i * U1 + u) * L, L)])
                a3 = jnp.maximum(a3, row[pl.ds((3 * QB + i * U1 + u) * L, L)])
            return (a0, a1, a2, a3)

        a0, a1, a2, a3 = lax.fori_loop(0, QB // U1, p1,
                                       (neg16, neg16, neg16, neg16))
        thr = jnp.min(jnp.minimum(jnp.minimum(a0, a1), jnp.minimum(a2, a3)))
        thr_s = _splat_f(thr)

        G = 8
        lim_s = _splat_i(CMAX + L)

        def p2(g, cntv):
            base = g * (G * L)
            xs = [row[pl.ds(base + k * L, L)] for k in range(G)]
            m01 = jnp.maximum(xs[0], xs[1])
            m23 = jnp.maximum(xs[2], xs[3])
            m45 = jnp.maximum(xs[4], xs[5])
            m67 = jnp.maximum(xs[6], xs[7])
            mx = jnp.maximum(jnp.maximum(m01, m23), jnp.maximum(m45, m67))
            has = jnp.any(mx >= thr_s)

            def taken(cntv):
                for k in range(G):
                    msk = xs[k] >= thr_s
                    pfx = plsc.cumsum(msk.astype(jnp.int32))
                    tgt = cntv + pfx - 1
                    ok = msk & (tgt < lim_s)
                    plsc.store_scatter(cval, [tgt], xs[k], mask=ok)
                    plsc.store_scatter(
                        cidx, [tgt], iota + (base + k * L), mask=ok)
                    cntv = cntv + plsc.all_reduce_population_count(msk)
                return cntv

            return lax.cond(has, taken, lambda z: z, cntv)

        cntv = lax.fori_loop(0, NV // G, p2, _splat_i(0))
        cnt = jnp.max(cntv)
        padbase = jnp.minimum(cnt, CMAX)
        for u in range(SU):
            cval[pl.ds(padbase + u * L, L)] = neg16

        def normal(_):
            _select_topk(
                val_load=lambda j: cval[pl.ds(j * L, L)],
                val_kill=lambda fsp: plsc.store_scatter(
                    cval, [fsp], neg16, mask=lane0),
                idx_of=lambda fsp: plsc.load_gather(cidx, [fsp]),
                ngroups=(cnt + SU * L - 1) // (SU * L),
                ovbuf=ovbuf, oibuf=oibuf, lane0=lane0, iota=iota)
            return 0

        def fallback(_):
            _select_topk(
                val_load=lambda j: row[pl.ds(j * L, L)],
                val_kill=lambda fsp: plsc.store_scatter(
                    row, [fsp], neg16, mask=lane0),
                idx_of=lambda fsp: fsp,
                ngroups=NV // SU,
                ovbuf=ovbuf, oibuf=oibuf, lane0=lane0, iota=iota)
            return 0

        lax.cond(cnt <= CMAX, normal, fallback, 0)

        pltpu.sync_copy(ovbuf, outv_hbm.at[base_row + r])
        pltpu.sync_copy(oibuf, outi_hbm.at[base_row + r])


@functools.lru_cache(maxsize=1)
def _topk_call():
    return functools.partial(
        pl.kernel,
        out_type=[
            jax.ShapeDtypeStruct((B, K), jnp.float32),
            jax.ShapeDtypeStruct((B, K), jnp.int32),
        ],
        mesh=plsc.VectorSubcoreMesh(core_axis_name="c", subcore_axis_name="s"),
        compiler_params=pltpu.CompilerParams(needs_layout_passes=False),
        scratch_types=[
            pltpu.VMEM((N,), jnp.float32),
            pltpu.VMEM((N,), jnp.float32),
            pltpu.VMEM((CMAX + SU * L,), jnp.float32),
            pltpu.VMEM((CMAX + SU * L,), jnp.int32),
            pltpu.VMEM((K,), jnp.float32),
            pltpu.VMEM((K,), jnp.int32),
            pltpu.SemaphoreType.DMA,
            pltpu.SemaphoreType.DMA,
        ],
    )(_topk_body)


def kernel(accumulated):
    topk_vals, topk_idx = _topk_call()(accumulated)
    return (topk_vals, topk_idx, accumulated)
